# Initial kernel scaffold; baseline (speedup 1.0000x reference)
import jax
import jax.numpy as jnp
from jax.experimental import pallas as pl


def kernel(w, c, negs, in_table, out_table):
    raise NotImplementedError("write your pallas kernel here")


# speedup vs baseline: 4.6562x; 4.6562x over previous
import functools

import jax
import jax.numpy as jnp
from jax import lax
from jax.experimental import pallas as pl
from jax.experimental.pallas import tpu as pltpu
from jax.experimental.pallas import tpu_sc as plsc

B_ = 16384
D_ = 64
NEG_ = 20
L_ = 16
NC_ = 2
NS_ = 16
NW_ = NC_ * NS_
CB_ = B_ // NW_
G_ = 32
NCHUNK_ = CB_ // G_
PAIRS_ = G_ * NEG_
IDXCAP_ = 128
NKSEG_ = PAIRS_ // IDXCAP_
NGRP_ = PAIRS_ // L_
KD_ = D_ // L_


def _sc_body(w_hbm, c_hbm, negs_hbm, in_hbm, out_hbm,
             pos_hbm, neg_hbm,
             idx_w, idx_c, idx_n, wi_v, wo_v, wn_v,
             pos_buf, neg_buf, sem):
    cid = lax.axis_index("c")
    sid = lax.axis_index("s")
    wid = sid * NC_ + cid
    base = wid * CB_
    nbase = wid * (CB_ * NEG_)

    pltpu.sync_copy(w_hbm.at[pl.ds(base, CB_)], idx_w)
    pltpu.sync_copy(c_hbm.at[pl.ds(base, CB_)], idx_c)
    pltpu.sync_copy(negs_hbm.at[pl.ds(nbase, CB_ * NEG_)], idx_n)

    iota = lax.iota(jnp.int32, L_)

    def chunk(g, carry):
        cps = [
            pltpu.async_copy(in_hbm.at[idx_w.at[pl.ds(g * G_, G_)]], wi_v, sem),
            pltpu.async_copy(out_hbm.at[idx_c.at[pl.ds(g * G_, G_)]], wo_v, sem),
        ]
        for k in range(NKSEG_):
            cps.append(pltpu.async_copy(
                out_hbm.at[idx_n.at[pl.ds(g * PAIRS_ + k * IDXCAP_, IDXCAP_)]],
                wn_v.at[pl.ds(k * IDXCAP_, IDXCAP_)], sem))
        for cp in cps:
            cp.wait()

        def elem(j, carry2):
            acc_neg, acc_pos = carry2
            wis = [wi_v[j, pl.ds(k * L_, L_)] for k in range(KD_)]
            pacc = wo_v[j, pl.ds(0, L_)] * wis[0]
            for k in range(1, KD_):
                pacc = pacc + wo_v[j, pl.ds(k * L_, L_)] * wis[k]
            lane_p = j & (L_ - 1)
            acc_pos = jnp.where(iota == lane_p, jnp.sum(pacc), acc_pos)
            pos_buf[pl.ds(g * G_ + j - lane_p, L_)] = acc_pos
            for n in range(NEG_):
                p = j * NEG_ + n
                a = wn_v[p, pl.ds(0, L_)] * wis[0]
                for k in range(1, KD_):
                    a = a + wn_v[p, pl.ds(k * L_, L_)] * wis[k]
                lane = p & (L_ - 1)
                acc_neg = jnp.where(iota == lane, jnp.sum(a), acc_neg)
                neg_buf[pl.ds(g * PAIRS_ + p - lane, L_)] = acc_neg
            return (acc_neg, acc_pos)
        zero = jnp.zeros((L_,), jnp.float32)
        lax.fori_loop(0, G_, elem, (zero, zero))
        return carry

    lax.fori_loop(0, NCHUNK_, chunk, 0)

    def sig_pos(i, c2):
        v = pos_buf[pl.ds(i * L_, L_)]
        pos_buf[pl.ds(i * L_, L_)] = 1.0 / (1.0 + jnp.exp(-v))
        return c2
    lax.fori_loop(0, CB_ // L_, sig_pos, 0)

    def sig_neg(i, c2):
        v = neg_buf[pl.ds(i * L_, L_)]
        neg_buf[pl.ds(i * L_, L_)] = 1.0 / (1.0 + jnp.exp(v))
        return c2
    lax.fori_loop(0, (CB_ * NEG_) // L_, sig_neg, 0)

    pltpu.sync_copy(pos_buf, pos_hbm.at[pl.ds(base, CB_)])
    pltpu.sync_copy(neg_buf, neg_hbm.at[pl.ds(nbase, CB_ * NEG_)])


_sc_call = functools.partial(
    pl.kernel,
    out_type=(
        jax.ShapeDtypeStruct((B_,), jnp.float32),
        jax.ShapeDtypeStruct((B_ * NEG_,), jnp.float32),
    ),
    mesh=plsc.VectorSubcoreMesh(core_axis_name="c", subcore_axis_name="s"),
    compiler_params=pltpu.CompilerParams(
        needs_layout_passes=False, use_tc_tiling_on_sc=False),
    scratch_types=[
        pltpu.VMEM((CB_,), jnp.int32),
        pltpu.VMEM((CB_,), jnp.int32),
        pltpu.VMEM((CB_ * NEG_,), jnp.int32),
        pltpu.VMEM((G_, D_), jnp.float32),
        pltpu.VMEM((G_, D_), jnp.float32),
        pltpu.VMEM((PAIRS_, D_), jnp.float32),
        pltpu.VMEM((CB_,), jnp.float32),
        pltpu.VMEM((CB_ * NEG_,), jnp.float32),
        pltpu.SemaphoreType.DMA,
    ],
)(_sc_body)


@jax.jit
def kernel(w, c, negs, in_table, out_table):
    w32 = w.astype(jnp.int32)
    c32 = c.astype(jnp.int32)
    negs_flat = negs.astype(jnp.int32).reshape(B_ * NEG_)
    pos_flat, neg_flat = _sc_call(w32, c32, negs_flat, in_table, out_table)
    return (pos_flat.reshape(B_, 1), neg_flat.reshape(B_, NEG_, 1))
